# baseline (speedup 1.0000x reference)
import dataclasses
import functools
import math

import jax
import jax.numpy as jnp
from jax.experimental import pallas as pl
from jax.experimental.pallas import tpu as pltpu
from jax.experimental.pallas import tpu_sc as plsc

_SC_MESH = plsc.VectorSubcoreMesh(core_axis_name="c", subcore_axis_name="s")
_SC_CP = pltpu.CompilerParams()
if "needs_layout_passes" in pltpu.CompilerParams.__dataclass_fields__:
    _SC_CP = dataclasses.replace(_SC_CP, needs_layout_passes=False)

_N_SUB = 32
_LANES = 16


def _gate_kernel(x_ref, w_ref, b_ref, idx_ref, score_ref, *, blk, n_expert):
    logits = jax.lax.dot_general(
        w_ref[...], x_ref[...], dimension_numbers=(((1,), (1,)), ((), ())),
        preferred_element_type=jnp.float32)
    logits = logits + b_ref[...]

    m = jnp.max(logits, axis=0, keepdims=True)
    denom = jnp.sum(jnp.exp(logits - m), axis=0, keepdims=True)
    score = 1.0 / denom

    srow = jax.lax.broadcasted_iota(jnp.int32, (n_expert, blk), 0)
    idx = jnp.min(jnp.where(logits == m, srow, n_expert), axis=0,
                  keepdims=True)

    idx_ref[...] = idx[None]
    score_ref[...] = score[None]


def _sc_hist_body(idx_hbm, hist_hbm, chunk, counts, sem, *, n_expert, cpt):
    c = jax.lax.axis_index("c")
    s = jax.lax.axis_index("s")
    sub = c * 16 + s
    pltpu.async_copy(idx_hbm.at[pl.ds(sub * cpt, cpt)], chunk, sem).wait()
    for g in range(n_expert // _LANES):
        counts[pl.ds(g * _LANES, _LANES)] = jnp.zeros((_LANES,), jnp.int32)
    for v in range(cpt // _LANES):
        iv = chunk[pl.ds(v * _LANES, _LANES)]
        cnt, last = plsc.scan_count(iv)
        prior = plsc.load_gather(counts, [iv])
        plsc.store_scatter(counts, [iv], prior + cnt, mask=last)
    pltpu.async_copy(counts, hist_hbm.at[sub], sem).wait()


def _sc_prune_body(idx_hbm, hist_hbm, out_hbm, chunk, outv, histv, counts,
                   sem, *, n_expert, cpt, capacity):
    c = jax.lax.axis_index("c")
    s = jax.lax.axis_index("s")
    sub = c * 16 + s
    cp_idx = pltpu.async_copy(idx_hbm.at[pl.ds(sub * cpt, cpt)], chunk, sem)
    pltpu.async_copy(hist_hbm, histv, sem).wait()
    cp_idx.wait()
    ngroups = n_expert // _LANES
    for g in range(ngroups):
        counts[pl.ds(g * _LANES, _LANES)] = jnp.zeros((_LANES,), jnp.int32)
    for r in range(_N_SUB):
        use = (r < sub)
        for g in range(ngroups):
            sl = pl.ds(g * _LANES, _LANES)
            counts[sl] += jnp.where(use, histv[r, sl], 0)
    for v in range(cpt // _LANES):
        iv = chunk[pl.ds(v * _LANES, _LANES)]
        cnt, last = plsc.scan_count(iv)
        prior = plsc.load_gather(counts, [iv])
        pos = prior + cnt - 1
        outv[pl.ds(v * _LANES, _LANES)] = jnp.where(pos < capacity, iv, -1)
        plsc.store_scatter(counts, [iv], prior + cnt, mask=last)
    pltpu.async_copy(outv, out_hbm.at[pl.ds(sub * cpt, cpt)], sem).wait()


@jax.jit
def kernel(inp, W, b):
    n, d = inp.shape
    e = W.shape[0]
    blk = 1024
    capacity = math.ceil(2.4 * n / e)
    grid = n // blk
    cpt = n // _N_SUB

    raw_idx, score_out = pl.pallas_call(
        functools.partial(_gate_kernel, blk=blk, n_expert=e),
        grid=(grid,),
        in_specs=[
            pl.BlockSpec((blk, d), lambda i: (i, 0)),
            pl.BlockSpec((e, d), lambda i: (0, 0)),
            pl.BlockSpec((e, 1), lambda i: (0, 0)),
        ],
        out_specs=[
            pl.BlockSpec((1, 1, blk), lambda i: (i, 0, 0)),
            pl.BlockSpec((1, 1, blk), lambda i: (i, 0, 0)),
        ],
        out_shape=[
            jax.ShapeDtypeStruct((grid, 1, blk), jnp.int32),
            jax.ShapeDtypeStruct((grid, 1, blk), jnp.float32),
        ],
    )(inp, W, b.reshape(e, 1))

    idx_flat = raw_idx.reshape(n)

    hist_k = pl.kernel(
        functools.partial(_sc_hist_body, n_expert=e, cpt=cpt),
        out_type=jax.ShapeDtypeStruct((_N_SUB, e), jnp.int32),
        mesh=_SC_MESH,
        scratch_types=[pltpu.VMEM((cpt,), jnp.int32),
                       pltpu.VMEM((e,), jnp.int32),
                       pltpu.SemaphoreType.DMA],
        compiler_params=_SC_CP,
    )
    hist = hist_k(idx_flat)

    prune_k = pl.kernel(
        functools.partial(_sc_prune_body, n_expert=e, cpt=cpt,
                          capacity=capacity),
        out_type=jax.ShapeDtypeStruct((n,), jnp.int32),
        mesh=_SC_MESH,
        scratch_types=[pltpu.VMEM((cpt,), jnp.int32),
                       pltpu.VMEM((cpt,), jnp.int32),
                       pltpu.VMEM((_N_SUB, e), jnp.int32),
                       pltpu.VMEM((e,), jnp.int32),
                       pltpu.SemaphoreType.DMA],
        compiler_params=_SC_CP,
    )
    pruned = prune_k(idx_flat, hist)

    return (pruned.reshape(n, 1), score_out.reshape(n, 1))


# speedup vs baseline: 1.5061x; 1.5061x over previous
import functools
import math

import jax
import jax.numpy as jnp
from jax.experimental import pallas as pl
from jax.experimental.pallas import tpu as pltpu


def _router_kernel(x_ref, w_ref, b_ref, idx_ref, score_ref, counts_ref,
                   *, blk, n_expert, capacity):
    step = pl.program_id(0)

    @pl.when(step == 0)
    def _init():
        counts_ref[...] = jnp.zeros_like(counts_ref)

    logits = jax.lax.dot_general(
        w_ref[...], x_ref[...], dimension_numbers=(((1,), (1,)), ((), ())),
        preferred_element_type=jnp.float32)
    logits = logits + b_ref[...]

    m = jnp.max(logits, axis=0, keepdims=True)
    denom = jnp.sum(jnp.exp(logits - m), axis=0, keepdims=True)
    score = 1.0 / denom

    srow = jax.lax.broadcasted_iota(jnp.int32, (n_expert, blk), 0)
    idx = jnp.min(jnp.where(logits == m, srow, n_expert), axis=0,
                  keepdims=True)
    onehot = (srow == idx).astype(jnp.float32)

    rj = jax.lax.broadcasted_iota(jnp.int32, (blk, blk), 0)
    ct = jax.lax.broadcasted_iota(jnp.int32, (blk, blk), 1)
    triu = (rj <= ct).astype(jnp.float32)
    cs = jax.lax.dot_general(
        onehot, triu, dimension_numbers=(((1,), (0,)), ((), ())),
        preferred_element_type=jnp.float32)

    prev = counts_ref[...]
    pos = jnp.sum((cs + prev) * onehot, axis=0, keepdims=True) - 1.0
    pruned = jnp.where(pos < capacity, idx, -1)

    counts_ref[...] = prev + jnp.sum(onehot, axis=1, keepdims=True)

    idx_ref[...] = pruned[None]
    score_ref[...] = score[None]


@jax.jit
def kernel(inp, W, b):
    n, d = inp.shape
    e = W.shape[0]
    blk = 1024
    capacity = math.ceil(2.4 * n / e)
    grid = n // blk

    idx_out, score_out = pl.pallas_call(
        functools.partial(_router_kernel, blk=blk, n_expert=e,
                          capacity=capacity),
        grid=(grid,),
        in_specs=[
            pl.BlockSpec((blk, d), lambda i: (i, 0)),
            pl.BlockSpec((e, d), lambda i: (0, 0)),
            pl.BlockSpec((e, 1), lambda i: (0, 0)),
        ],
        out_specs=[
            pl.BlockSpec((1, 1, blk), lambda i: (i, 0, 0)),
            pl.BlockSpec((1, 1, blk), lambda i: (i, 0, 0)),
        ],
        out_shape=[
            jax.ShapeDtypeStruct((grid, 1, blk), jnp.int32),
            jax.ShapeDtypeStruct((grid, 1, blk), jnp.float32),
        ],
        scratch_shapes=[pltpu.VMEM((e, 1), jnp.float32)],
    )(inp, W, b.reshape(e, 1))
    return (idx_out.reshape(n, 1), score_out.reshape(n, 1))
